# Initial kernel scaffold; baseline (speedup 1.0000x reference)
import functools

import jax
import jax.numpy as jnp
from jax import lax
from jax.experimental import pallas as pl
from jax.experimental.pallas import tpu as pltpu
from jax.experimental.pallas import tpu_sc as plsc

N = 10000
E = 320000
D = 128
DE = 16
EPS = 1e-5

NC = 2
NS = 16
NW = NC * NS
EPT = E // NW
K = 80
NCHUNK = EPT // K

ROWS_PER_TILE = N // NS
ZROWS = 125

NB = 2000
EB = 3200


def _prep_body(x_ref, wa_ref, ba_ref, wsx_ref, bsx_ref, a_ref, s_ref):
    x = x_ref[...]
    a_ref[...] = jnp.dot(x, wa_ref[...], preferred_element_type=jnp.float32) + ba_ref[...]
    s_ref[...] = jnp.dot(x, wsx_ref[...], preferred_element_type=jnp.float32) + bsx_ref[...]


def _prep(x, wa, ba, wsx, bsx):
    grid = (N // NB,)
    return pl.pallas_call(
        _prep_body,
        grid=grid,
        in_specs=[
            pl.BlockSpec((NB, D), lambda i: (i, 0)),
            pl.BlockSpec((D, D), lambda i: (0, 0)),
            pl.BlockSpec((1, D), lambda i: (0, 0)),
            pl.BlockSpec((D, 2 * D), lambda i: (0, 0)),
            pl.BlockSpec((1, 2 * D), lambda i: (0, 0)),
        ],
        out_specs=[
            pl.BlockSpec((NB, D), lambda i: (i, 0)),
            pl.BlockSpec((NB, 2 * D), lambda i: (i, 0)),
        ],
        out_shape=[
            jax.ShapeDtypeStruct((N, D), jnp.float32),
            jax.ShapeDtypeStruct((N, 2 * D), jnp.float32),
        ],
    )(x, wa, ba, wsx, bsx)


def _sc_gather_body(a_hbm, s_hbm, src_hbm, dst_hbm, adst_hbm, ssrc_hbm,
                    sidx, didx, arows, srows, sem):
    cid = lax.axis_index("c")
    sid = lax.axis_index("s")
    wid = sid * NC + cid
    base = wid * EPT
    pltpu.sync_copy(src_hbm.at[pl.ds(base, EPT)], sidx)
    pltpu.sync_copy(dst_hbm.at[pl.ds(base, EPT)], didx)

    @pl.loop(0, EPT, step=K)
    def _chunk(off):
        pltpu.async_copy(s_hbm.at[sidx.at[pl.ds(off, K)]], srows, sem).wait()
        pltpu.async_copy(a_hbm.at[didx.at[pl.ds(off, K)]], arows, sem).wait()
        pltpu.sync_copy(srows, ssrc_hbm.at[pl.ds(base + off, K)])
        pltpu.sync_copy(arows, adst_hbm.at[pl.ds(base + off, K)])


def _sc_gather(a, s, src, dst):
    mesh = plsc.VectorSubcoreMesh(core_axis_name="c", subcore_axis_name="s")
    return pl.kernel(
        _sc_gather_body,
        out_type=[
            jax.ShapeDtypeStruct((E, D), jnp.float32),
            jax.ShapeDtypeStruct((E, 2 * D), jnp.float32),
        ],
        mesh=mesh,
        scratch_types=[
            pltpu.VMEM((EPT,), jnp.int32),
            pltpu.VMEM((EPT,), jnp.int32),
            pltpu.VMEM((K, D), jnp.float32),
            pltpu.VMEM((K, 2 * D), jnp.float32),
            pltpu.SemaphoreType.DMA,
        ],
    )(a, s, src, dst)


def _edge_body(adst_ref, ssrc_ref, ef_ref, ewn_ref, wge_ref, bns_ref, bnb_ref,
               wg2_ref, b2_ref, msg_ref):
    adst = adst_ref[...]
    ssrc = ssrc_ref[...]
    ef = ef_ref[...]
    h1 = adst + ssrc[:, :D] + jnp.dot(ef, wge_ref[...], preferred_element_type=jnp.float32)
    h1 = h1 * bns_ref[...] + bnb_ref[...]
    h1 = 0.5 * h1 * (1.0 + lax.erf(h1 * 0.7071067811865476))
    logit = jnp.dot(h1, wg2_ref[...], preferred_element_type=jnp.float32) + b2_ref[...]
    gate = 1.0 / (1.0 + jnp.exp(-logit[:, 0]))
    coef = ewn_ref[0, 0, :] * gate
    msg_ref[...] = ssrc[:, D:] * coef[:, None]


def _edge(adst, ssrc, ef, ewn3, wge, bns, bnb, wg2, b2):
    grid = (E // EB,)
    return pl.pallas_call(
        _edge_body,
        grid=grid,
        in_specs=[
            pl.BlockSpec((EB, D), lambda i: (i, 0)),
            pl.BlockSpec((EB, 2 * D), lambda i: (i, 0)),
            pl.BlockSpec((EB, DE), lambda i: (i, 0)),
            pl.BlockSpec((1, 1, EB), lambda i: (i, 0, 0)),
            pl.BlockSpec((DE, D), lambda i: (0, 0)),
            pl.BlockSpec((1, D), lambda i: (0, 0)),
            pl.BlockSpec((1, D), lambda i: (0, 0)),
            pl.BlockSpec((D, 1), lambda i: (0, 0)),
            pl.BlockSpec((1, 1), lambda i: (0, 0)),
        ],
        out_specs=pl.BlockSpec((EB, D), lambda i: (i, 0)),
        out_shape=jax.ShapeDtypeStruct((E, D), jnp.float32),
    )(adst, ssrc, ef, ewn3, wge, bns, bnb, wg2, b2)


def _sc_scatter_body(msg_hbm, dst_hbm, aggp_hbm, agg_sh, zbuf, rows, didx, sem):
    cid = lax.axis_index("c")
    sid = lax.axis_index("s")

    zero = jnp.zeros((16,), jnp.float32)

    @pl.loop(0, ZROWS)
    def _zr(r):
        for j in range(D // 16):
            zbuf[r, pl.ds(j * 16, 16)] = zero

    @pl.loop(0, ROWS_PER_TILE, step=ZROWS)
    def _zc(r0):
        pltpu.sync_copy(zbuf, agg_sh.at[pl.ds(sid * ROWS_PER_TILE + r0, ZROWS)])

    plsc.subcore_barrier()

    base = cid * (E // NC) + sid * EPT

    @pl.loop(0, EPT, step=K)
    def _chunk(off):
        pltpu.sync_copy(msg_hbm.at[pl.ds(base + off, K)], rows)
        pltpu.sync_copy(dst_hbm.at[pl.ds(base + off, K)], didx)
        pltpu.sync_copy(rows, agg_sh.at[didx], add=True)

    plsc.subcore_barrier()

    @pl.loop(0, ROWS_PER_TILE, step=ZROWS)
    def _out(r0):
        r = sid * ROWS_PER_TILE + r0
        pltpu.sync_copy(agg_sh.at[pl.ds(r, ZROWS)], aggp_hbm.at[cid, pl.ds(r, ZROWS)])


def _sc_scatter(msg, dst):
    mesh = plsc.VectorSubcoreMesh(core_axis_name="c", subcore_axis_name="s")
    return pl.kernel(
        _sc_scatter_body,
        out_type=jax.ShapeDtypeStruct((NC, N, D), jnp.float32),
        mesh=mesh,
        scratch_types=[
            pltpu.VMEM_SHARED((N, D), jnp.float32),
            pltpu.VMEM((ZROWS, D), jnp.float32),
            pltpu.VMEM((K, D), jnp.float32),
            pltpu.VMEM((K,), jnp.int32),
            pltpu.SemaphoreType.DMA,
        ],
    )(msg, dst)


def _final_body(x_ref, aggp_ref, wux_ref, wua_ref, bu_ref, lng_ref, lnb_ref, o_ref):
    x = x_ref[...]
    agg = aggp_ref[0] + aggp_ref[1]
    h = (jnp.dot(x, wux_ref[...], preferred_element_type=jnp.float32)
         + jnp.dot(agg, wua_ref[...], preferred_element_type=jnp.float32)
         + bu_ref[...])
    h = 0.5 * h * (1.0 + lax.erf(h * 0.7071067811865476))
    r = x + h
    mu = jnp.mean(r, axis=-1, keepdims=True)
    var = jnp.mean((r - mu) ** 2, axis=-1, keepdims=True)
    o_ref[...] = (r - mu) / jnp.sqrt(var + EPS) * lng_ref[...] + lnb_ref[...]


def _final(x, aggp, wux, wua, bu, lng, lnb):
    grid = (N // NB,)
    return pl.pallas_call(
        _final_body,
        grid=grid,
        in_specs=[
            pl.BlockSpec((NB, D), lambda i: (i, 0)),
            pl.BlockSpec((NC, NB, D), lambda i: (0, i, 0)),
            pl.BlockSpec((D, D), lambda i: (0, 0)),
            pl.BlockSpec((D, D), lambda i: (0, 0)),
            pl.BlockSpec((1, D), lambda i: (0, 0)),
            pl.BlockSpec((1, D), lambda i: (0, 0)),
            pl.BlockSpec((1, D), lambda i: (0, 0)),
        ],
        out_specs=pl.BlockSpec((NB, D), lambda i: (i, 0)),
        out_shape=jax.ShapeDtypeStruct((N, D), jnp.float32),
    )(x, aggp, wux, wua, bu, lng, lnb)


def kernel(x, edge_index, edge_weight_norm, edge_feat, batch, W_phi, b_phi,
           W_g1, b_g1, bn_gamma, bn_beta, W_g2, b_g2, W_u, b_u, ln_gamma, ln_beta):
    src = edge_index[0]
    dst = edge_index[1]

    wa = W_g1[:D]
    wsx = jnp.concatenate([W_g1[D:2 * D], W_phi], axis=1)
    ba = b_g1.reshape(1, D)
    bsx = jnp.concatenate([jnp.zeros((D,), jnp.float32), b_phi]).reshape(1, 2 * D)

    a_tab, s_tab = _prep(x, wa, ba, wsx, bsx)
    adst, ssrc = _sc_gather(a_tab, s_tab, src, dst)

    bns = (bn_gamma / jnp.sqrt(1.0 + EPS)).reshape(1, D)
    bnb = bn_beta.reshape(1, D)
    ewn3 = edge_weight_norm.reshape(E // EB, 1, EB)
    msg = _edge(adst, ssrc, edge_feat, ewn3, W_g1[2 * D:], bns, bnb,
                W_g2, b_g2.reshape(1, 1))

    aggp = _sc_scatter(msg, dst)

    return _final(x, aggp, W_u[:D], W_u[D:], b_u.reshape(1, D),
                  ln_gamma.reshape(1, D), ln_beta.reshape(1, D))


# speedup vs baseline: 2.9255x; 2.9255x over previous
import functools

import jax
import jax.numpy as jnp
from jax import lax
from jax.experimental import pallas as pl
from jax.experimental.pallas import tpu as pltpu
from jax.experimental.pallas import tpu_sc as plsc

N = 10000
E = 320000
D = 128
DE = 16
EPS = 1e-5

NC = 2
NS = 16
NW = NC * NS
EPT = E // NW
K = 80
NCHUNK = EPT // K

HALF = N // NC
HPAD = HALF + 8
SLAB = 312
SLAB_LAST = 320

NB = 2000
NBF = 1000
EB = 3200


def _prep_body(x_ref, wa_ref, ba_ref, wsx_ref, bsx_ref, a_ref, s_ref):
    x = x_ref[...]
    a_ref[...] = jnp.dot(x, wa_ref[...], preferred_element_type=jnp.float32) + ba_ref[...]
    s_ref[...] = jnp.dot(x, wsx_ref[...], preferred_element_type=jnp.float32) + bsx_ref[...]


def _prep(x, wa, ba, wsx, bsx):
    grid = (N // NB,)
    return pl.pallas_call(
        _prep_body,
        grid=grid,
        in_specs=[
            pl.BlockSpec((NB, D), lambda i: (i, 0)),
            pl.BlockSpec((D, D), lambda i: (0, 0)),
            pl.BlockSpec((1, D), lambda i: (0, 0)),
            pl.BlockSpec((D, 2 * D), lambda i: (0, 0)),
            pl.BlockSpec((1, 2 * D), lambda i: (0, 0)),
        ],
        out_specs=[
            pl.BlockSpec((NB, D), lambda i: (i, 0)),
            pl.BlockSpec((NB, 2 * D), lambda i: (i, 0)),
        ],
        out_shape=[
            jax.ShapeDtypeStruct((N, D), jnp.float32),
            jax.ShapeDtypeStruct((N, 2 * D), jnp.float32),
        ],
    )(x, wa, ba, wsx, bsx)


def _sc_gather_body(a_hbm, s_hbm, src_hbm, dst_hbm, adst_hbm, ssrc_hbm,
                    sidx, didx, arows, srows, sem):
    cid = lax.axis_index("c")
    sid = lax.axis_index("s")
    wid = sid * NC + cid
    base = wid * EPT
    pltpu.sync_copy(src_hbm.at[pl.ds(base, EPT)], sidx)
    pltpu.sync_copy(dst_hbm.at[pl.ds(base, EPT)], didx)

    @pl.loop(0, EPT, step=K)
    def _chunk(off):
        pltpu.async_copy(s_hbm.at[sidx.at[pl.ds(off, K)]], srows, sem).wait()
        pltpu.async_copy(a_hbm.at[didx.at[pl.ds(off, K)]], arows, sem).wait()
        pltpu.sync_copy(srows, ssrc_hbm.at[pl.ds(base + off, K)])
        pltpu.sync_copy(arows, adst_hbm.at[pl.ds(base + off, K)])


def _sc_gather(a, s, src, dst):
    mesh = plsc.VectorSubcoreMesh(core_axis_name="c", subcore_axis_name="s")
    return pl.kernel(
        _sc_gather_body,
        out_type=[
            jax.ShapeDtypeStruct((E, D), jnp.float32),
            jax.ShapeDtypeStruct((E, 2 * D), jnp.float32),
        ],
        mesh=mesh,
        scratch_types=[
            pltpu.VMEM((EPT,), jnp.int32),
            pltpu.VMEM((EPT,), jnp.int32),
            pltpu.VMEM((K, D), jnp.float32),
            pltpu.VMEM((K, 2 * D), jnp.float32),
            pltpu.SemaphoreType.DMA,
        ],
    )(a, s, src, dst)


def _edge_body(adst_ref, ssrc_ref, ef_ref, ewn_ref, wge_ref, bns_ref, bnb_ref,
               wg2_ref, b2_ref, msg_ref):
    adst = adst_ref[...]
    ssrc = ssrc_ref[...]
    ef = ef_ref[...]
    h1 = adst + ssrc[:, :D] + jnp.dot(ef, wge_ref[...], preferred_element_type=jnp.float32)
    h1 = h1 * bns_ref[...] + bnb_ref[...]
    h1 = 0.5 * h1 * (1.0 + lax.erf(h1 * 0.7071067811865476))
    logit = jnp.dot(h1, wg2_ref[...], preferred_element_type=jnp.float32) + b2_ref[...]
    gate = 1.0 / (1.0 + jnp.exp(-logit[:, 0]))
    coef = ewn_ref[0, 0, :] * gate
    msg_ref[...] = ssrc[:, D:] * coef[:, None]


def _edge(adst, ssrc, ef, ewn3, wge, bns, bnb, wg2, b2):
    grid = (E // EB,)
    return pl.pallas_call(
        _edge_body,
        grid=grid,
        in_specs=[
            pl.BlockSpec((EB, D), lambda i: (i, 0)),
            pl.BlockSpec((EB, 2 * D), lambda i: (i, 0)),
            pl.BlockSpec((EB, DE), lambda i: (i, 0)),
            pl.BlockSpec((1, 1, EB), lambda i: (i, 0, 0)),
            pl.BlockSpec((DE, D), lambda i: (0, 0)),
            pl.BlockSpec((1, D), lambda i: (0, 0)),
            pl.BlockSpec((1, D), lambda i: (0, 0)),
            pl.BlockSpec((D, 1), lambda i: (0, 0)),
            pl.BlockSpec((1, 1), lambda i: (0, 0)),
        ],
        out_specs=pl.BlockSpec((EB, D), lambda i: (i, 0)),
        out_shape=jax.ShapeDtypeStruct((E, D), jnp.float32),
    )(adst, ssrc, ef, ewn3, wge, bns, bnb, wg2, b2)


def _sc_scatter_body(msg_hbm, dst_hbm, aggp_hbm, agg_sh, zbuf, rows, didx, sem):
    cid = lax.axis_index("c")
    sid = lax.axis_index("s")

    zero = jnp.zeros((16,), jnp.float32)

    @pl.loop(0, SLAB_LAST)
    def _zr(r):
        for j in range(D // 16):
            zbuf[r, pl.ds(j * 16, 16)] = zero

    r0 = sid * SLAB

    @pl.when(sid < NS - 1)
    def _z0():
        pltpu.sync_copy(zbuf.at[pl.ds(0, SLAB)], agg_sh.at[pl.ds(r0, SLAB)])

    @pl.when(sid == NS - 1)
    def _z1():
        pltpu.sync_copy(zbuf, agg_sh.at[pl.ds(r0, SLAB_LAST)])
        pltpu.sync_copy(zbuf.at[pl.ds(0, 8)], agg_sh.at[pl.ds(HALF, 8)])

    plsc.subcore_barrier()

    ept2 = E // NS
    base = sid * ept2
    nbase = cid * HALF

    @pl.loop(0, ept2, step=K)
    def _chunk(off):
        pltpu.sync_copy(msg_hbm.at[pl.ds(base + off, K)], rows)
        pltpu.sync_copy(dst_hbm.at[pl.ds(base + off, K)], didx)
        for j in range(K // 16):
            v = didx[pl.ds(j * 16, 16)] - nbase
            ok = (v >= 0) & (v < HALF)
            didx[pl.ds(j * 16, 16)] = jnp.where(ok, v, HALF)
        pltpu.sync_copy(rows, agg_sh.at[didx], add=True)

    plsc.subcore_barrier()

    @pl.when(sid < NS - 1)
    def _o0():
        pltpu.sync_copy(agg_sh.at[pl.ds(r0, SLAB)],
                        aggp_hbm.at[cid, pl.ds(r0, SLAB)])

    @pl.when(sid == NS - 1)
    def _o1():
        pltpu.sync_copy(agg_sh.at[pl.ds(r0, SLAB_LAST)],
                        aggp_hbm.at[cid, pl.ds(r0, SLAB_LAST)])


def _sc_scatter(msg, dst):
    mesh = plsc.VectorSubcoreMesh(core_axis_name="c", subcore_axis_name="s")
    return pl.kernel(
        _sc_scatter_body,
        out_type=jax.ShapeDtypeStruct((NC, HALF, D), jnp.float32),
        mesh=mesh,
        scratch_types=[
            pltpu.VMEM_SHARED((HPAD, D), jnp.float32),
            pltpu.VMEM((SLAB_LAST, D), jnp.float32),
            pltpu.VMEM((K, D), jnp.float32),
            pltpu.VMEM((K,), jnp.int32),
            pltpu.SemaphoreType.DMA,
        ],
    )(msg, dst)


def _final_body(x_ref, aggp_ref, wux_ref, wua_ref, bu_ref, lng_ref, lnb_ref, o_ref):
    x = x_ref[...]
    agg = aggp_ref[0]
    h = (jnp.dot(x, wux_ref[...], preferred_element_type=jnp.float32)
         + jnp.dot(agg, wua_ref[...], preferred_element_type=jnp.float32)
         + bu_ref[...])
    h = 0.5 * h * (1.0 + lax.erf(h * 0.7071067811865476))
    r = x + h
    mu = jnp.mean(r, axis=-1, keepdims=True)
    var = jnp.mean((r - mu) ** 2, axis=-1, keepdims=True)
    o_ref[...] = (r - mu) / jnp.sqrt(var + EPS) * lng_ref[...] + lnb_ref[...]


def _final(x, aggp, wux, wua, bu, lng, lnb):
    grid = (N // NBF,)
    return pl.pallas_call(
        _final_body,
        grid=grid,
        in_specs=[
            pl.BlockSpec((NBF, D), lambda i: (i, 0)),
            pl.BlockSpec((1, NBF, D), lambda i: (i // 5, i % 5, 0)),
            pl.BlockSpec((D, D), lambda i: (0, 0)),
            pl.BlockSpec((D, D), lambda i: (0, 0)),
            pl.BlockSpec((1, D), lambda i: (0, 0)),
            pl.BlockSpec((1, D), lambda i: (0, 0)),
            pl.BlockSpec((1, D), lambda i: (0, 0)),
        ],
        out_specs=pl.BlockSpec((NBF, D), lambda i: (i, 0)),
        out_shape=jax.ShapeDtypeStruct((N, D), jnp.float32),
    )(x, aggp, wux, wua, bu, lng, lnb)


def kernel(x, edge_index, edge_weight_norm, edge_feat, batch, W_phi, b_phi,
           W_g1, b_g1, bn_gamma, bn_beta, W_g2, b_g2, W_u, b_u, ln_gamma, ln_beta):
    src = edge_index[0]
    dst = edge_index[1]

    wa = W_g1[:D]
    wsx = jnp.concatenate([W_g1[D:2 * D], W_phi], axis=1)
    ba = b_g1.reshape(1, D)
    bsx = jnp.concatenate([jnp.zeros((D,), jnp.float32), b_phi]).reshape(1, 2 * D)

    a_tab, s_tab = _prep(x, wa, ba, wsx, bsx)
    adst, ssrc = _sc_gather(a_tab, s_tab, src, dst)

    bns = (bn_gamma / jnp.sqrt(1.0 + EPS)).reshape(1, D)
    bnb = bn_beta.reshape(1, D)
    ewn3 = edge_weight_norm.reshape(E // EB, 1, EB)
    msg = _edge(adst, ssrc, edge_feat, ewn3, W_g1[2 * D:], bns, bnb,
                W_g2, b_g2.reshape(1, 1))

    aggp = _sc_scatter(msg, dst)

    return _final(x, aggp, W_u[:D], W_u[D:], b_u.reshape(1, D),
                  ln_gamma.reshape(1, D), ln_beta.reshape(1, D))
